# Initial kernel scaffold; baseline (speedup 1.0000x reference)
import jax
import jax.numpy as jnp
from jax.experimental import pallas as pl


def kernel(x, edge_index, edge_attr, We1, be1, Wl1, bl1, g1, b1, We2, be2, Wl2, bl2, g2, b2, Wout, bout):
    raise NotImplementedError("write your pallas kernel here")


# speedup vs baseline: 3.6240x; 3.6240x over previous
import functools

import jax
import jax.numpy as jnp
from jax import lax
from jax.experimental import pallas as pl
from jax.experimental.pallas import tpu as pltpu
from jax.experimental.pallas import tpu_sc as plsc

N = 10000
D = 256
HALF = 128
E = 160000
ED = 16

NC = 2
NS = 16
K = 64
NPH = 2
NCHP = 80
NCH = NPH * NCHP
EPT = K * NCH
EPAD = EPT * NS
NP = 10240
RPT = NP // NS

KD = 64
NCHD = 80
EPTD = KD * NCHD


def _sc_spmm_body(table, srcm, dstm, wm, s_out,
                  gidx, dstv, wv, rows_a, rows_b, accum, gs_a, gs_b):
    c = lax.axis_index("c")
    s = lax.axis_index("s")

    pltpu.sync_copy(wm.at[s], wv)

    off = jnp.full((16,), c * N, dtype=jnp.int32)

    zv = jnp.zeros((16,), jnp.float32)

    def zr(i, carry):
        for g in range(HALF // 16):
            rows_a[i, pl.ds(g * 16, 16)] = zv
        return carry
    lax.fori_loop(0, K, zr, 0)

    base = s * RPT
    for q in range(RPT // K):
        pltpu.sync_copy(rows_a, accum.at[pl.ds(base + q * K, K)])

    plsc.subcore_barrier()

    def gather(i, buf, sem):
        return pltpu.make_async_copy(table.at[gidx.at[i]], buf, sem)

    def scale(ibase, buf):
        wrow = ibase // 128
        wcol = ibase % 128

        def pg(g, carry):
            wch = wv[wrow, pl.ds(wcol + g * 16, 16)]
            row0 = g * 16
            for l in range(16):
                wvec = jnp.full((16,), wch[l], dtype=jnp.float32)
                for q in range(HALF // 16):
                    sl = pl.ds(q * 16, 16)
                    buf[row0 + l, sl] = buf[row0 + l, sl] * wvec
            return carry
        lax.fori_loop(0, K // 16, pg, 0)

    def scatter(i, buf):
        pltpu.sync_copy(buf, accum.at[dstv.at[i]], add=True)

    npair = NCHP // 2
    for ph in range(NPH):
        pltpu.sync_copy(srcm.at[s, ph], gidx)
        pltpu.sync_copy(dstm.at[s, ph], dstv)

        def adj(i, carry):
            for g in range(K // 16):
                sl = pl.ds(g * 16, 16)
                gidx[i, sl] = gidx[i, sl] + off
            return carry
        lax.fori_loop(0, NCHP, adj, 0)

        gather(0, rows_a, gs_a).start()
        ph_ebase = ph * NCHP * K

        def mbody(m, carry):
            i0 = 2 * m
            i1 = i0 + 1
            gather(i0, rows_a, gs_a).wait()
            gather(i1, rows_b, gs_b).start()
            scale(ph_ebase + i0 * K, rows_a)
            scatter(i0, rows_a)
            gather(i1, rows_b, gs_b).wait()

            @pl.when(m < npair - 1)
            def _():
                gather(i0 + 2, rows_a, gs_a).start()
            scale(ph_ebase + i1 * K, rows_b)
            scatter(i1, rows_b)
            return carry
        lax.fori_loop(0, npair, mbody, 0)

    plsc.subcore_barrier()

    pltpu.sync_copy(accum.at[pl.ds(base, RPT)],
                    s_out.at[c, pl.ds(base, RPT)])


_sc_spmm = pl.kernel(
    _sc_spmm_body,
    out_type=jax.ShapeDtypeStruct((NC, NP, HALF), jnp.float32),
    mesh=plsc.VectorSubcoreMesh(core_axis_name="c", subcore_axis_name="s"),
    scratch_types=(
        pltpu.VMEM((NCHP, K), jnp.int32),
        pltpu.VMEM((NCHP, K), jnp.int32),
        pltpu.VMEM((EPT // 128, 128), jnp.float32),
        pltpu.VMEM((K, HALF), jnp.float32),
        pltpu.VMEM((K, HALF), jnp.float32),
        pltpu.VMEM_SHARED((NP, HALF), jnp.float32),
        pltpu.SemaphoreType.DMA,
        pltpu.SemaphoreType.DMA,
    ),
)


def _sc_deg_body(dstm, w1m, w2m, deg_out, dstv, wv, w2v, degbuf, dega):
    c = lax.axis_index("c")
    s = lax.axis_index("s")

    pltpu.sync_copy(dstm.at[c, s], dstv)
    pltpu.sync_copy(w1m.at[c, s], wv)
    pltpu.sync_copy(w2m.at[c, s], w2v)

    zv = jnp.zeros((16,), jnp.float32)

    def zd(i, carry):
        for g in range(128 // 16):
            degbuf[i, pl.ds(g * 16, 16)] = zv
        return carry
    lax.fori_loop(0, KD, zd, 0)

    base = s * RPT
    for q in range(RPT // KD):
        pltpu.sync_copy(degbuf, dega.at[pl.ds(base + q * KD, KD)])

    plsc.subcore_barrier()

    lane = lax.iota(jnp.int32, 16)
    m0 = lane == 0
    m1 = lane == 1

    def mbody(i, carry):
        ibase = i * KD
        wrow = ibase // 128
        wcol = ibase % 128

        def dg(g, carry2):
            w1c = wv[wrow, pl.ds(wcol + g * 16, 16)]
            w2c = w2v[wrow, pl.ds(wcol + g * 16, 16)]
            for l in range(16):
                row = jnp.where(
                    m0, jnp.full((16,), w1c[l], jnp.float32),
                    jnp.where(m1, jnp.full((16,), w2c[l], jnp.float32), zv))
                degbuf[g * 16 + l, pl.ds(0, 16)] = row
            return carry2
        lax.fori_loop(0, KD // 16, dg, 0)
        pltpu.sync_copy(degbuf, dega.at[dstv.at[i]], add=True)
        return carry
    lax.fori_loop(0, NCHD, mbody, 0)

    plsc.subcore_barrier()
    pltpu.sync_copy(dega.at[pl.ds(base, RPT)],
                    deg_out.at[c, pl.ds(base, RPT)])


_sc_deg = pl.kernel(
    _sc_deg_body,
    out_type=jax.ShapeDtypeStruct((NC, NP, 128), jnp.float32),
    mesh=plsc.VectorSubcoreMesh(core_axis_name="c", subcore_axis_name="s"),
    scratch_types=(
        pltpu.VMEM((NCHD, KD), jnp.int32),
        pltpu.VMEM((EPTD // 128, 128), jnp.float32),
        pltpu.VMEM((EPTD // 128, 128), jnp.float32),
        pltpu.VMEM((KD, 128), jnp.float32),
        pltpu.VMEM_SHARED((NP, 128), jnp.float32),
    ),
)


_EBLK = 6400


def _ew_body(attr_ref, wet_ref, bet_ref, out_ref):
    z = lax.dot_general(wet_ref[...], attr_ref[...],
                        (((1,), (1,)), ((), ())),
                        preferred_element_type=jnp.float32)
    out_ref[...] = jax.nn.softplus(z + bet_ref[...])


def _edge_weights(edge_attr, wet, bet):
    grid = E // _EBLK
    return pl.pallas_call(
        _ew_body,
        grid=(grid,),
        in_specs=[
            pl.BlockSpec((_EBLK, ED), lambda i: (i, 0)),
            pl.BlockSpec((2, ED), lambda i: (0, 0)),
            pl.BlockSpec((2, 1), lambda i: (0, 0)),
        ],
        out_specs=pl.BlockSpec((2, _EBLK), lambda i: (0, i)),
        out_shape=jax.ShapeDtypeStruct((2, E), jnp.float32),
    )(edge_attr, wet, bet)


_RBLK = 1000


def _layer_body(col, final, s_ref, deg_ref, hin_ref, wl_ref, bl_ref,
                g_ref, b_ref, *rest):
    if final:
        wout_ref, bout_ref, out_ref = rest
    else:
        (out_ref,) = rest
    sm = jnp.concatenate([s_ref[0], s_ref[1]], axis=1)
    hin = jnp.concatenate([hin_ref[0], hin_ref[1]], axis=1)
    degw = (deg_ref[0, :, col] + deg_ref[1, :, col])[:, None]
    aggr = sm - degw * hin
    t = jnp.dot(aggr, wl_ref[...], preferred_element_type=jnp.float32)
    t = jnp.maximum(t + bl_ref[...], 0.0)
    mu = jnp.mean(t, axis=1, keepdims=True)
    var = jnp.mean((t - mu) ** 2, axis=1, keepdims=True)
    t = (t - mu) * lax.rsqrt(var + 1e-5) * g_ref[...] + b_ref[...]
    h = t + hin
    if final:
        out_ref[...] = jnp.dot(h, wout_ref[...],
                               preferred_element_type=jnp.float32) + bout_ref[...]
    else:
        out_ref[...] = jnp.stack([h[:, :HALF], h[:, HALF:]])


def _layer_tc(sm, deg, col, hin, wl, bl, g, b, wout=None, bout=None):
    final = wout is not None
    grid = N // _RBLK
    in_specs = [
        pl.BlockSpec((NC, _RBLK, HALF), lambda i: (0, i, 0)),
        pl.BlockSpec((NC, _RBLK, 128), lambda i: (0, i, 0)),
        pl.BlockSpec((NC, _RBLK, HALF), lambda i: (0, i, 0)),
        pl.BlockSpec((D, D), lambda i: (0, 0)),
        pl.BlockSpec((1, D), lambda i: (0, 0)),
        pl.BlockSpec((1, D), lambda i: (0, 0)),
        pl.BlockSpec((1, D), lambda i: (0, 0)),
    ]
    args = [sm, deg, hin, wl, bl, g, b]
    if final:
        in_specs += [pl.BlockSpec((D, D), lambda i: (0, 0)),
                     pl.BlockSpec((1, D), lambda i: (0, 0))]
        args += [wout, bout]
        out_spec = pl.BlockSpec((_RBLK, D), lambda i: (i, 0))
        out_shape = jax.ShapeDtypeStruct((N, D), jnp.float32)
    else:
        out_spec = pl.BlockSpec((NC, _RBLK, HALF), lambda i: (0, i, 0))
        out_shape = jax.ShapeDtypeStruct((NC, N, HALF), jnp.float32)
    return pl.pallas_call(
        functools.partial(_layer_body, col, final),
        grid=(grid,),
        in_specs=in_specs,
        out_specs=out_spec,
        out_shape=out_shape,
    )(*args)


def kernel(x, edge_index, edge_attr, We1, be1, Wl1, bl1, g1, b1,
           We2, be2, Wl2, bl2, g2, b2, Wout, bout):
    wet = jnp.concatenate([We1, We2], axis=1).T
    bet = jnp.concatenate([be1, be2]).reshape(2, 1)
    w12 = _edge_weights(edge_attr, wet, bet)

    pad = EPAD - E
    src_p = jnp.pad(edge_index[0], (0, pad)).astype(jnp.int32)
    dst_p = jnp.pad(edge_index[1], (0, pad)).astype(jnp.int32)
    w1_p = jnp.pad(w12[0], (0, pad))
    w2_p = jnp.pad(w12[1], (0, pad))

    srcm = src_p.reshape(NS, NPH, NCHP, K)
    dstm = dst_p.reshape(NS, NPH, NCHP, K)
    w1m = w1_p.reshape(NS, EPT // 128, 128)
    w2m = w2_p.reshape(NS, EPT // 128, 128)

    dstm_d = dst_p.reshape(NC, NS, NCHD, KD)
    w1m_d = w1_p.reshape(NC, NS, EPTD // 128, 128)
    w2m_d = w2_p.reshape(NC, NS, EPTD // 128, 128)

    x2 = x.reshape(N, NC, HALF).transpose(1, 0, 2)

    deg = _sc_deg(dstm_d, w1m_d, w2m_d)
    s1 = _sc_spmm(x2.reshape(NC * N, HALF), srcm, dstm, w1m)
    h1 = _layer_tc(s1, deg, 0, x2, Wl1, bl1.reshape(1, D),
                   g1.reshape(1, D), b1.reshape(1, D))
    s2 = _sc_spmm(h1.reshape(NC * N, HALF), srcm, dstm, w2m)
    out = _layer_tc(s2, deg, 1, h1, Wl2, bl2.reshape(1, D),
                    g2.reshape(1, D), b2.reshape(1, D),
                    Wout, bout.reshape(1, D))
    return out


# speedup vs baseline: 3.8776x; 1.0700x over previous
import functools

import jax
import jax.numpy as jnp
from jax import lax
from jax.experimental import pallas as pl
from jax.experimental.pallas import tpu as pltpu
from jax.experimental.pallas import tpu_sc as plsc

N = 10000
D = 256
HALF = 128
E = 160000
ED = 16

NC = 2
NS = 16
K = 128
NPH = 4
NCHP = 20
NCH = NPH * NCHP
EPT = K * NCH
EPAD = EPT * NS
NP = 10240
RPT = NP // NS

KD = 64
NCHD = 80
EPTD = KD * NCHD


def _sc_spmm_body(table, srcm, dstm, wm, s_out,
                  gidx, dstv, wv, rows_a, rows_b, accum,
                  gs_a, gs_b, ss_a, ss_b):
    c = lax.axis_index("c")
    s = lax.axis_index("s")

    pltpu.sync_copy(wm.at[s], wv)

    off = jnp.full((16,), c * N, dtype=jnp.int32)

    zv = jnp.zeros((16,), jnp.float32)

    def zr(i, carry):
        for g in range(HALF // 16):
            rows_a[i, pl.ds(g * 16, 16)] = zv
        return carry
    lax.fori_loop(0, K, zr, 0)

    base = s * RPT
    for q in range(RPT // K):
        pltpu.sync_copy(rows_a, accum.at[pl.ds(base + q * K, K)])

    plsc.subcore_barrier()

    def gather(i, buf, sem):
        return pltpu.make_async_copy(table.at[gidx.at[i]], buf, sem)

    def scale(ibase, buf):
        wrow = ibase // 128
        wcol = ibase % 128

        def pg(g, carry):
            wch = wv[wrow, pl.ds(wcol + g * 16, 16)]
            row0 = g * 16
            for l in range(16):
                wvec = jnp.full((16,), wch[l], dtype=jnp.float32)
                for q in range(HALF // 16):
                    sl = pl.ds(q * 16, 16)
                    buf[row0 + l, sl] = buf[row0 + l, sl] * wvec
            return carry
        lax.fori_loop(0, K // 16, pg, 0)

    def scatter_start(i, buf, sem):
        pltpu.async_copy(buf, accum.at[dstv.at[i]], sem, add=True)

    def scatter_wait(i, buf, sem):
        pltpu.make_async_copy(buf, accum.at[dstv.at[i]], sem).wait()

    npair = NCHP // 2
    for ph in range(NPH):
        pltpu.sync_copy(srcm.at[s, ph], gidx)
        pltpu.sync_copy(dstm.at[s, ph], dstv)

        def adj(i, carry):
            for g in range(K // 16):
                sl = pl.ds(g * 16, 16)
                gidx[i, sl] = gidx[i, sl] + off
            return carry
        lax.fori_loop(0, NCHP, adj, 0)

        gather(0, rows_a, gs_a).start()
        ph_ebase = ph * NCHP * K

        def mbody(m, carry):
            i0 = 2 * m
            i1 = i0 + 1
            gather(i0, rows_a, gs_a).wait()

            @pl.when(m > 0)
            def _():
                scatter_wait(i1 - 2, rows_b, ss_b)
            gather(i1, rows_b, gs_b).start()
            scale(ph_ebase + i0 * K, rows_a)
            scatter_start(i0, rows_a, ss_a)
            gather(i1, rows_b, gs_b).wait()
            scatter_wait(i0, rows_a, ss_a)

            @pl.when(m < npair - 1)
            def _():
                gather(i0 + 2, rows_a, gs_a).start()
            scale(ph_ebase + i1 * K, rows_b)
            scatter_start(i1, rows_b, ss_b)
            return carry
        lax.fori_loop(0, npair, mbody, 0)
        scatter_wait(NCHP - 1, rows_b, ss_b)

    plsc.subcore_barrier()

    pltpu.sync_copy(accum.at[pl.ds(base, RPT)],
                    s_out.at[c, pl.ds(base, RPT)])


_sc_spmm = pl.kernel(
    _sc_spmm_body,
    out_type=jax.ShapeDtypeStruct((NC, NP, HALF), jnp.float32),
    mesh=plsc.VectorSubcoreMesh(core_axis_name="c", subcore_axis_name="s"),
    scratch_types=(
        pltpu.VMEM((NCHP, K), jnp.int32),
        pltpu.VMEM((NCHP, K), jnp.int32),
        pltpu.VMEM((EPT // 128, 128), jnp.float32),
        pltpu.VMEM((K, HALF), jnp.float32),
        pltpu.VMEM((K, HALF), jnp.float32),
        pltpu.VMEM_SHARED((NP, HALF), jnp.float32),
        pltpu.SemaphoreType.DMA,
        pltpu.SemaphoreType.DMA,
        pltpu.SemaphoreType.DMA,
        pltpu.SemaphoreType.DMA,
    ),
)


def _sc_deg_body(dstm, w1m, w2m, deg_out, dstv, wv, w2v, degbuf, dega):
    c = lax.axis_index("c")
    s = lax.axis_index("s")

    pltpu.sync_copy(dstm.at[c, s], dstv)
    pltpu.sync_copy(w1m.at[c, s], wv)
    pltpu.sync_copy(w2m.at[c, s], w2v)

    zv = jnp.zeros((16,), jnp.float32)

    def zd(i, carry):
        for g in range(128 // 16):
            degbuf[i, pl.ds(g * 16, 16)] = zv
        return carry
    lax.fori_loop(0, KD, zd, 0)

    base = s * RPT
    for q in range(RPT // KD):
        pltpu.sync_copy(degbuf, dega.at[pl.ds(base + q * KD, KD)])

    plsc.subcore_barrier()

    lane = lax.iota(jnp.int32, 16)
    m0 = lane == 0
    m1 = lane == 1

    def mbody(i, carry):
        ibase = i * KD
        wrow = ibase // 128
        wcol = ibase % 128

        def dg(g, carry2):
            w1c = wv[wrow, pl.ds(wcol + g * 16, 16)]
            w2c = w2v[wrow, pl.ds(wcol + g * 16, 16)]
            for l in range(16):
                row = jnp.where(
                    m0, jnp.full((16,), w1c[l], jnp.float32),
                    jnp.where(m1, jnp.full((16,), w2c[l], jnp.float32), zv))
                degbuf[g * 16 + l, pl.ds(0, 16)] = row
            return carry2
        lax.fori_loop(0, KD // 16, dg, 0)
        pltpu.sync_copy(degbuf, dega.at[dstv.at[i]], add=True)
        return carry
    lax.fori_loop(0, NCHD, mbody, 0)

    plsc.subcore_barrier()
    pltpu.sync_copy(dega.at[pl.ds(base, RPT)],
                    deg_out.at[c, pl.ds(base, RPT)])


_sc_deg = pl.kernel(
    _sc_deg_body,
    out_type=jax.ShapeDtypeStruct((NC, NP, 128), jnp.float32),
    mesh=plsc.VectorSubcoreMesh(core_axis_name="c", subcore_axis_name="s"),
    scratch_types=(
        pltpu.VMEM((NCHD, KD), jnp.int32),
        pltpu.VMEM((EPTD // 128, 128), jnp.float32),
        pltpu.VMEM((EPTD // 128, 128), jnp.float32),
        pltpu.VMEM((KD, 128), jnp.float32),
        pltpu.VMEM_SHARED((NP, 128), jnp.float32),
    ),
)


_EBLK = 6400


def _ew_body(attr_ref, wet_ref, bet_ref, out_ref):
    z = lax.dot_general(wet_ref[...], attr_ref[...],
                        (((1,), (1,)), ((), ())),
                        preferred_element_type=jnp.float32)
    out_ref[...] = jax.nn.softplus(z + bet_ref[...])


def _edge_weights(edge_attr, wet, bet):
    grid = E // _EBLK
    return pl.pallas_call(
        _ew_body,
        grid=(grid,),
        in_specs=[
            pl.BlockSpec((_EBLK, ED), lambda i: (i, 0)),
            pl.BlockSpec((2, ED), lambda i: (0, 0)),
            pl.BlockSpec((2, 1), lambda i: (0, 0)),
        ],
        out_specs=pl.BlockSpec((2, _EBLK), lambda i: (0, i)),
        out_shape=jax.ShapeDtypeStruct((2, E), jnp.float32),
    )(edge_attr, wet, bet)


_RBLK = 1000


def _layer_body(col, final, s_ref, deg_ref, hin_ref, wl_ref, bl_ref,
                g_ref, b_ref, *rest):
    if final:
        wout_ref, bout_ref, out_ref = rest
    else:
        (out_ref,) = rest
    sm = jnp.concatenate([s_ref[0], s_ref[1]], axis=1)
    hin = jnp.concatenate([hin_ref[0], hin_ref[1]], axis=1)
    degw = (deg_ref[0, :, col] + deg_ref[1, :, col])[:, None]
    aggr = sm - degw * hin
    t = jnp.dot(aggr, wl_ref[...], preferred_element_type=jnp.float32)
    t = jnp.maximum(t + bl_ref[...], 0.0)
    mu = jnp.mean(t, axis=1, keepdims=True)
    var = jnp.mean((t - mu) ** 2, axis=1, keepdims=True)
    t = (t - mu) * lax.rsqrt(var + 1e-5) * g_ref[...] + b_ref[...]
    h = t + hin
    if final:
        out_ref[...] = jnp.dot(h, wout_ref[...],
                               preferred_element_type=jnp.float32) + bout_ref[...]
    else:
        out_ref[...] = jnp.stack([h[:, :HALF], h[:, HALF:]])


def _layer_tc(sm, deg, col, hin, wl, bl, g, b, wout=None, bout=None):
    final = wout is not None
    grid = N // _RBLK
    in_specs = [
        pl.BlockSpec((NC, _RBLK, HALF), lambda i: (0, i, 0)),
        pl.BlockSpec((NC, _RBLK, 128), lambda i: (0, i, 0)),
        pl.BlockSpec((NC, _RBLK, HALF), lambda i: (0, i, 0)),
        pl.BlockSpec((D, D), lambda i: (0, 0)),
        pl.BlockSpec((1, D), lambda i: (0, 0)),
        pl.BlockSpec((1, D), lambda i: (0, 0)),
        pl.BlockSpec((1, D), lambda i: (0, 0)),
    ]
    args = [sm, deg, hin, wl, bl, g, b]
    if final:
        in_specs += [pl.BlockSpec((D, D), lambda i: (0, 0)),
                     pl.BlockSpec((1, D), lambda i: (0, 0))]
        args += [wout, bout]
        out_spec = pl.BlockSpec((_RBLK, D), lambda i: (i, 0))
        out_shape = jax.ShapeDtypeStruct((N, D), jnp.float32)
    else:
        out_spec = pl.BlockSpec((NC, _RBLK, HALF), lambda i: (0, i, 0))
        out_shape = jax.ShapeDtypeStruct((NC, N, HALF), jnp.float32)
    return pl.pallas_call(
        functools.partial(_layer_body, col, final),
        grid=(grid,),
        in_specs=in_specs,
        out_specs=out_spec,
        out_shape=out_shape,
    )(*args)


def kernel(x, edge_index, edge_attr, We1, be1, Wl1, bl1, g1, b1,
           We2, be2, Wl2, bl2, g2, b2, Wout, bout):
    wet = jnp.concatenate([We1, We2], axis=1).T
    bet = jnp.concatenate([be1, be2]).reshape(2, 1)
    w12 = _edge_weights(edge_attr, wet, bet)

    pad = EPAD - E
    src_p = jnp.pad(edge_index[0], (0, pad)).astype(jnp.int32)
    dst_p = jnp.pad(edge_index[1], (0, pad)).astype(jnp.int32)
    w1_p = jnp.pad(w12[0], (0, pad))
    w2_p = jnp.pad(w12[1], (0, pad))

    srcm = src_p.reshape(NS, NPH, NCHP, K)
    dstm = dst_p.reshape(NS, NPH, NCHP, K)
    w1m = w1_p.reshape(NS, EPT // 128, 128)
    w2m = w2_p.reshape(NS, EPT // 128, 128)

    dstm_d = dst_p.reshape(NC, NS, NCHD, KD)
    w1m_d = w1_p.reshape(NC, NS, EPTD // 128, 128)
    w2m_d = w2_p.reshape(NC, NS, EPTD // 128, 128)

    x2 = x.reshape(N, NC, HALF).transpose(1, 0, 2)

    deg = _sc_deg(dstm_d, w1m_d, w2m_d)
    s1 = _sc_spmm(x2.reshape(NC * N, HALF), srcm, dstm, w1m)
    h1 = _layer_tc(s1, deg, 0, x2, Wl1, bl1.reshape(1, D),
                   g1.reshape(1, D), b1.reshape(1, D))
    s2 = _sc_spmm(h1.reshape(NC * N, HALF), srcm, dstm, w2m)
    out = _layer_tc(s2, deg, 1, h1, Wl2, bl2.reshape(1, D),
                    g2.reshape(1, D), b2.reshape(1, D),
                    Wout, bout.reshape(1, D))
    return out
